# baseline (speedup 1.0000x reference)
import functools

import jax
import jax.numpy as jnp
from jax import lax
from jax.experimental import pallas as pl
from jax.experimental.pallas import tpu as pltpu
from jax.experimental.pallas import tpu_sc as plsc

B = 1024
D = 768
K = 100000

KBLK = 2000
NKB = K // KBLK

_I32_MAX = jnp.iinfo(jnp.int32).max


CB = 512
NCH = B // CB


def _argmin_body(emb_ref, cmd_ref, idx_ref, val_scr, idx_scr, embt_scr):
    kb = pl.program_id(0)

    @pl.when(kb == 0)
    def _init():
        embt_scr[...] = emb_ref[...].T
        val_scr[...] = jnp.full((1, B), jnp.inf, jnp.float32)
        idx_scr[...] = jnp.zeros((1, B), jnp.int32)

    cmd = cmd_ref[...]
    c_sq = 0.5 * jnp.sum(cmd * cmd, axis=1, keepdims=True)
    rowf = lax.broadcasted_iota(jnp.int32, (KBLK, 1), 0).astype(jnp.float32)
    base = kb * KBLK

    def _mm(j):
        return lax.dot_general(
            cmd, embt_scr[:, j * CB:(j + 1) * CB], (((1,), (0,)), ((), ())),
            preferred_element_type=jnp.float32)

    dots = _mm(0)
    for j in range(NCH):
        sl = slice(j * CB, (j + 1) * CB)
        score = c_sq - dots
        if j + 1 < NCH:
            dots = _mm(j + 1)
        blk_min = jnp.min(score, axis=0, keepdims=True)
        blk_idxf = jnp.min(
            jnp.where(score == blk_min, rowf, jnp.inf), axis=0, keepdims=True)
        better = blk_min < val_scr[:, sl]
        val_scr[:, sl] = jnp.where(better, blk_min, val_scr[:, sl])
        idx_scr[:, sl] = jnp.where(
            better, base + blk_idxf.astype(jnp.int32), idx_scr[:, sl])

    @pl.when(kb == NKB - 1)
    def _out():
        idx_ref[...] = idx_scr[...]


def _nearest_idx(embed, command_embeds):
    idx2d = pl.pallas_call(
        _argmin_body,
        grid=(NKB,),
        in_specs=[
            pl.BlockSpec((B, D), lambda k: (0, 0)),
            pl.BlockSpec((KBLK, D), lambda k: (k, 0)),
        ],
        out_specs=pl.BlockSpec((1, B), lambda k: (0, 0)),
        out_shape=jax.ShapeDtypeStruct((1, B), jnp.int32),
        scratch_shapes=[
            pltpu.VMEM((1, B), jnp.float32),
            pltpu.VMEM((1, B), jnp.int32),
            pltpu.VMEM((D, B), jnp.float32),
        ],
    )(embed, command_embeds)
    return idx2d.reshape(B)


_NC = 2
_NS = 16
_NW = _NC * _NS
_BPW = B // _NW


@functools.cache
def _sc_gather():
    @functools.partial(
        pl.kernel,
        mesh=plsc.VectorSubcoreMesh(core_axis_name="c", subcore_axis_name="s"),
        out_type=jax.ShapeDtypeStruct((B, D), jnp.float32),
        scratch_types=[
            pltpu.VMEM((_BPW,), jnp.int32),
            pltpu.VMEM((_BPW, D), jnp.float32),
            pltpu.SemaphoreType.DMA,
        ],
    )
    def gather(table_hbm, idx_hbm, out_hbm, idx_v, rows_v, sem):
        wid = lax.axis_index("s") * _NC + lax.axis_index("c")
        base = wid * _BPW
        pltpu.sync_copy(idx_hbm.at[pl.ds(base, _BPW)], idx_v)
        pltpu.async_copy(table_hbm.at[idx_v], rows_v, sem).wait()
        pltpu.sync_copy(rows_v, out_hbm.at[pl.ds(base, _BPW)])

    return gather


def kernel(embed, command_embeds):
    idx = _nearest_idx(embed, command_embeds)
    return _sc_gather()(command_embeds, idx)


# speedup vs baseline: 1.0462x; 1.0462x over previous
import functools

import jax
import jax.numpy as jnp
from jax import lax
from jax.experimental import pallas as pl
from jax.experimental.pallas import tpu as pltpu
from jax.experimental.pallas import tpu_sc as plsc

B = 1024
D = 768
K = 100000

KBLK = 1000
NKB = K // KBLK

_I32_MAX = jnp.iinfo(jnp.int32).max


CB = 512
NCH = B // CB


def _argmin_body(emb_ref, cmd_ref, idx_ref, val_scr, idx_scr, embt_scr):
    kb = pl.program_id(0)

    @pl.when(kb == 0)
    def _init():
        embt_scr[...] = emb_ref[...].T
        val_scr[...] = jnp.full((1, B), jnp.inf, jnp.float32)
        idx_scr[...] = jnp.zeros((1, B), jnp.int32)

    cmd = cmd_ref[...]
    c_sq = 0.5 * jnp.sum(cmd * cmd, axis=1, keepdims=True)
    rowf = lax.broadcasted_iota(jnp.int32, (KBLK, 1), 0).astype(jnp.float32)
    base = kb * KBLK

    def _mm(j):
        return lax.dot_general(
            cmd, embt_scr[:, j * CB:(j + 1) * CB], (((1,), (0,)), ((), ())),
            preferred_element_type=jnp.float32)

    dots = _mm(0)
    for j in range(NCH):
        sl = slice(j * CB, (j + 1) * CB)
        score = c_sq - dots
        if j + 1 < NCH:
            dots = _mm(j + 1)
        blk_min = jnp.min(score, axis=0, keepdims=True)
        blk_idxf = jnp.min(
            jnp.where(score == blk_min, rowf, jnp.inf), axis=0, keepdims=True)
        better = blk_min < val_scr[:, sl]
        val_scr[:, sl] = jnp.where(better, blk_min, val_scr[:, sl])
        idx_scr[:, sl] = jnp.where(
            better, base + blk_idxf.astype(jnp.int32), idx_scr[:, sl])

    @pl.when(kb == NKB - 1)
    def _out():
        idx_ref[...] = idx_scr[...]


def _nearest_idx(embed, command_embeds):
    idx2d = pl.pallas_call(
        _argmin_body,
        grid=(NKB,),
        in_specs=[
            pl.BlockSpec((B, D), lambda k: (0, 0)),
            pl.BlockSpec((KBLK, D), lambda k: (k, 0)),
        ],
        out_specs=pl.BlockSpec((1, B), lambda k: (0, 0)),
        out_shape=jax.ShapeDtypeStruct((1, B), jnp.int32),
        scratch_shapes=[
            pltpu.VMEM((1, B), jnp.float32),
            pltpu.VMEM((1, B), jnp.int32),
            pltpu.VMEM((D, B), jnp.float32),
        ],
    )(embed, command_embeds)
    return idx2d.reshape(B)


_NC = 2
_NS = 16
_NW = _NC * _NS
_BPW = B // _NW


@functools.cache
def _sc_gather():
    @functools.partial(
        pl.kernel,
        mesh=plsc.VectorSubcoreMesh(core_axis_name="c", subcore_axis_name="s"),
        out_type=jax.ShapeDtypeStruct((B, D), jnp.float32),
        scratch_types=[
            pltpu.VMEM((_BPW,), jnp.int32),
            pltpu.VMEM((_BPW, D), jnp.float32),
            pltpu.SemaphoreType.DMA,
        ],
    )
    def gather(table_hbm, idx_hbm, out_hbm, idx_v, rows_v, sem):
        wid = lax.axis_index("s") * _NC + lax.axis_index("c")
        base = wid * _BPW
        pltpu.sync_copy(idx_hbm.at[pl.ds(base, _BPW)], idx_v)
        pltpu.async_copy(table_hbm.at[idx_v], rows_v, sem).wait()
        pltpu.sync_copy(rows_v, out_hbm.at[pl.ds(base, _BPW)])

    return gather


def kernel(embed, command_embeds):
    idx = _nearest_idx(embed, command_embeds)
    return _sc_gather()(command_embeds, idx)
